# baseline (speedup 1.0000x reference)
import functools

import jax
import jax.numpy as jnp
from jax import lax
from jax.experimental import pallas as pl
from jax.experimental.pallas import tpu as pltpu
from jax.experimental.pallas import tpu_sc as plsc

VOCAB = 1000000
MAXLEN = 200
EMBED = 32
BATCH = 4096

NC = 2
NS = 16
NW = NC * NS
L = 16

TOK_W = BATCH * MAXLEN // NW
GROUP = 512
NG = TOK_W // GROUP
CHUNK = 128
NCK = GROUP // CHUNK
NGB = 4
OUT_G = GROUP * EMBED


def _iota16():
    return lax.broadcasted_iota(jnp.int32, (L,), 0)


def _body(x_hbm, tok_hbm, pos_hbm, out_hbm, xv0, xv1, ix0, ix1, gbuf,
          ob0, ob1, posv, xsems, gsems, osems):
    wid = lax.axis_index("s") * NC + lax.axis_index("c")
    xbase = wid * TOK_W
    XV = [xv0, xv1]
    IX = [ix0, ix1]
    OB = [ob0, ob1]

    pltpu.sync_copy(pos_hbm, posv)

    def xdma(g, r):
        return pltpu.make_async_copy(
            x_hbm.at[pl.ds(xbase + g * GROUP, GROUP)], XV[r], xsems[r])

    def gather(r, c, v):
        return pltpu.make_async_copy(
            tok_hbm.at[IX[r].at[pl.ds(c * CHUNK, CHUNK)]], gbuf.at[v],
            gsems[v])

    def odma(g, r):
        return pltpu.make_async_copy(
            OB[r], out_hbm.at[pl.ds((wid * NG + g) * OUT_G, OUT_G)],
            osems[r])

    def compute_idxg(r):
        def ibody(i, c):
            sl = pl.ds(i * L, L)
            IX[r][sl] = lax.shift_right_logical(XV[r][sl], 2)
            return c
        lax.fori_loop(0, GROUP // L, ibody, 0)

    def extract_chunk(v, c, r, g):
        def slab(s, cc):
            base = c * CHUNK + s * L
            tok = XV[r][pl.ds(base, L)]
            rowloc = _iota16() + s * L
            col = (tok & 3) << 5
            grow = _iota16() + base
            pf = lax.rem(grow + g * GROUP, MAXLEN) * EMBED
            of = grow * EMBED
            for m in range(EMBED):
                t = plsc.load_gather(gbuf.at[v], [rowloc, col])
                p = plsc.load_gather(posv, [pf])
                plsc.store_scatter(OB[r], [of], t + p)
                if m < EMBED - 1:
                    col = col + 1
                    pf = pf + 1
                    of = of + 1
            return cc
        lax.fori_loop(0, CHUNK // L, slab, 0)

    xdma(0, 0).start()
    xdma(0, 0).wait()
    compute_idxg(0)
    xdma(1, 1).start()
    for v in range(NGB):
        gather(0, v, v).start()

    def outer(t, carry):
        for r in range(2):
            g = t * 2 + r

            @pl.when(g < NG - 1)
            def _():
                xdma(g + 1, r ^ 1).wait()
                compute_idxg(r ^ 1)

            @pl.when(g >= 2)
            def _():
                odma(g - 2, r).wait()

            for c in range(NCK):
                gather(r, c, c).wait()
                extract_chunk(c, c, r, g)

                @pl.when(g < NG - 1)
                def _():
                    gather(r ^ 1, c, c).start()

            odma(g, r).start()

            @pl.when(g < NG - 2)
            def _():
                xdma(g + 2, r).start()
        return carry

    lax.fori_loop(0, NG // 2, outer, 0)

    odma(NG - 2, (NG - 2) % 2).wait()
    odma(NG - 1, (NG - 1) % 2).wait()


@jax.jit
def kernel(x, token_table, pos_table):
    x1 = x.reshape(BATCH * MAXLEN).astype(jnp.int32)
    tok2 = token_table.reshape(VOCAB * EMBED // 128, 128)
    pos1 = pos_table.reshape(MAXLEN * EMBED)
    mesh = plsc.VectorSubcoreMesh(
        core_axis_name="c", subcore_axis_name="s", num_cores=NC, num_subcores=NS
    )
    run = pl.kernel(
        _body,
        out_type=jax.ShapeDtypeStruct((BATCH * MAXLEN * EMBED,), jnp.float32),
        mesh=mesh,
        scratch_types=[
            pltpu.VMEM((GROUP,), jnp.int32),
            pltpu.VMEM((GROUP,), jnp.int32),
            pltpu.VMEM((GROUP,), jnp.int32),
            pltpu.VMEM((GROUP,), jnp.int32),
            pltpu.VMEM((NGB, CHUNK, 128), jnp.float32),
            pltpu.VMEM((OUT_G,), jnp.float32),
            pltpu.VMEM((OUT_G,), jnp.float32),
            pltpu.VMEM((MAXLEN * EMBED,), jnp.float32),
            [pltpu.SemaphoreType.DMA] * 2,
            [pltpu.SemaphoreType.DMA] * NGB,
            [pltpu.SemaphoreType.DMA] * 2,
        ],
        compiler_params=pltpu.CompilerParams(needs_layout_passes=False),
    )
    out = run(x1, tok2, pos1)
    return out.reshape(BATCH, MAXLEN, EMBED)


# speedup vs baseline: 2.4388x; 2.4388x over previous
import functools

import jax
import jax.numpy as jnp
from jax import lax
from jax.experimental import pallas as pl
from jax.experimental.pallas import tpu as pltpu
from jax.experimental.pallas import tpu_sc as plsc

VOCAB = 1000000
MAXLEN = 200
EMBED = 32
BATCH = 4096

NC = 2
NS = 16
NW = NC * NS
L = 16

SEQ_PER_W = BATCH // NW
NCH = 2
CH = MAXLEN // NCH
NB = 4
NT = SEQ_PER_W // NB
ROWS = MAXLEN * EMBED // 128


def _body(x_hbm, tok_hbm, pos_hbm, out_hbm, idx_v, pos_v, gbuf, obuf,
          gsems, osems):
    wid = lax.axis_index("s") * NC + lax.axis_index("c")

    pltpu.sync_copy(x_hbm.at[wid], idx_v)
    pltpu.sync_copy(pos_hbm, pos_v)

    def issue_gather(k, b):
        for j in range(NCH):
            pltpu.async_copy(
                tok_hbm.at[idx_v.at[k].at[j]], gbuf.at[b].at[j], gsems[b]
            )

    def wait_gather(k, b):
        for j in range(NCH):
            pltpu.make_async_copy(
                tok_hbm.at[idx_v.at[k].at[j]], gbuf.at[b].at[j], gsems[b]
            ).wait()

    def out_slice(k):
        return out_hbm.at[pl.ds((wid * SEQ_PER_W + k) * ROWS, ROWS)]

    for b in range(NB):
        issue_gather(b, b)

    def outer(t, carry):
        for b in range(NB):
            k = t * NB + b
            wait_gather(k, b)

            @pl.when(t > 0)
            def _():
                pltpu.make_async_copy(obuf.at[b], out_slice(k - NB),
                                      osems[b]).wait()

            def add_body(q, c):
                for j in range(NCH):
                    for rr in range(4):
                        for h in range(EMBED // L):
                            o = pl.ds((rr * EMBED + h * L) % 128, L)
                            g = pl.ds(h * L, L)
                            obuf[b, j * (CH // 4) + q, o] = (
                                gbuf[b, j, q * 4 + rr, g]
                                + pos_v[j * (CH // 4) + q, o]
                            )
                return c

            lax.fori_loop(0, CH // 4, add_body, 0)
            pltpu.async_copy(obuf.at[b], out_slice(k), osems[b])

            @pl.when(t < NT - 1)
            def _():
                issue_gather(k + NB, b)
        return carry

    lax.fori_loop(0, NT, outer, 0)

    for b in range(NB):
        pltpu.make_async_copy(obuf.at[b], out_slice((NT - 1) * NB + b),
                              osems[b]).wait()


@jax.jit
def kernel(x, token_table, pos_table):
    x4 = x.reshape(NW, SEQ_PER_W, NCH, CH).astype(jnp.int32)
    pos2 = pos_table.reshape(ROWS, 128)
    mesh = plsc.VectorSubcoreMesh(
        core_axis_name="c", subcore_axis_name="s", num_cores=NC, num_subcores=NS
    )
    run = pl.kernel(
        _body,
        out_type=jax.ShapeDtypeStruct((BATCH * ROWS, 128), jnp.float32),
        mesh=mesh,
        scratch_types=[
            pltpu.VMEM((SEQ_PER_W, NCH, CH), jnp.int32),
            pltpu.VMEM((ROWS, 128), jnp.float32),
            pltpu.VMEM((NB, NCH, CH, EMBED), jnp.float32),
            pltpu.VMEM((NB, ROWS, 128), jnp.float32),
            [pltpu.SemaphoreType.DMA] * NB,
            [pltpu.SemaphoreType.DMA] * NB,
        ],
        compiler_params=pltpu.CompilerParams(use_tc_tiling_on_sc=False),
    )
    out = run(x4, token_table, pos2)
    return out.reshape(BATCH, MAXLEN, EMBED)
